# baseline (speedup 1.0000x reference)
import functools

import jax
import jax.numpy as jnp
from jax import lax
from jax.experimental import pallas as pl
from jax.experimental.pallas import tpu as pltpu
from jax.experimental.pallas import tpu_sc as plsc

_N = 10000
_F_IN = 128
_H = 3
_C = 64
_HC = _H * _C
_SEM = 128
_E = 160000
_ROW = 72
_K = 80
_NT = 16
_EPT = _E // _NT
_NCH = _EPT // _K
_RPT = 624
_MMB = 1000
_PBB = 1000


def _mm_body(x_ref, wl_ref, wr_ref, *out_refs):
    x = x_ref[...]
    xl = jnp.dot(x, wl_ref[...], preferred_element_type=jnp.float32)
    xr = jnp.dot(x, wr_ref[...], preferred_element_type=jnp.float32)
    for h in range(_H):
        out_refs[h][...] = xl[:, h * _C:(h + 1) * _C]
        out_refs[_H + h][...] = xr[:, h * _C:(h + 1) * _C]


def _project(x, Wl, Wr):
    return pl.pallas_call(
        _mm_body,
        grid=(_N // _MMB,),
        in_specs=[
            pl.BlockSpec((_MMB, _F_IN), lambda i: (i, 0)),
            pl.BlockSpec((_F_IN, _HC), lambda i: (0, 0)),
            pl.BlockSpec((_F_IN, _HC), lambda i: (0, 0)),
        ],
        out_specs=[pl.BlockSpec((_MMB, _C), lambda i: (i, 0))] * (2 * _H),
        out_shape=[jax.ShapeDtypeStruct((_N, _C), jnp.float32)] * (2 * _H),
    )(x, Wl, Wr)


def _acc_slices():
    out = []
    off = 0
    while off < _RPT:
        r = min(_K, _RPT - off)
        out.append((off, r))
        off += r
    return out


_PF = jnp.int32
_BREV = [0, 8, 4, 12, 2, 10, 6, 14, 1, 9, 5, 13, 3, 11, 7, 15]


def _edge_body(ei0, ei1, xl0, xl1, xl2, xr0, xr1, xr2, att_hbm, out_hbm,
               srcb, dstb, xlb0, xlb1, xrb0, xrb1, scb0, scb1,
               att_v, acc, sl0, sl1, sr0, sr1, ss0, ss1):
    cid = lax.axis_index("c")
    sid = lax.axis_index("s")
    xls = (xl0, xl1, xl2)
    xrs = (xr0, xr1, xr2)
    xlbs = (xlb0, xlb1)
    xrbs = (xrb0, xrb1)
    scbs = (scb0, scb1)
    sls = (sl0, sl1)
    srs = (sr0, sr1)
    sss = (ss0, ss1)

    pltpu.sync_copy(att_hbm, att_v)

    @pl.when(cid == 0)
    def _():
        pltpu.sync_copy(ei0.at[0, sid], srcb)
        pltpu.sync_copy(ei0.at[1, sid], dstb)

    @pl.when(cid != 0)
    def _():
        pltpu.sync_copy(ei1.at[0, sid], srcb)
        pltpu.sync_copy(ei1.at[1, sid], dstb)

    base0 = sid * _RPT
    lanes = lax.iota(jnp.int32, 16)

    def issue(i, h, par):
        pltpu.async_copy(xls[h].at[srcb.at[i]], xlbs[par], sls[par])
        pltpu.async_copy(xrs[h].at[dstb.at[i]], xrbs[par], srs[par])

    def wait_gather(i, h, par):
        pltpu.make_async_copy(xls[h].at[srcb.at[i]], xlbs[par],
                              sls[par]).wait()
        pltpu.make_async_copy(xrs[h].at[dstb.at[i]], xrbs[par],
                              srs[par]).wait()

    def scatter(i, par):
        pltpu.async_copy(scbs[par], acc.at[dstb.at[i]], sss[par], add=True)

    def wait_scatter(i, par):
        pltpu.make_async_copy(scbs[par], acc.at[dstb.at[i]],
                              sss[par]).wait()

    for h in range(_H):
        att_vecs = [att_v[pl.ds(h * _C + k * 16, 16)] for k in range(_C // 16)]

        def _zero_row16(r, _):
            for scb in scbs:
                for c in range(4):
                    scb[r, pl.ds(c * 16, 16)] = jnp.zeros((16,), jnp.float32)
                scb[r, pl.ds(56, 16)] = jnp.zeros((16,), jnp.float32)
            return 0
        lax.fori_loop(0, _K, _zero_row16, 0)

        for off, r in _acc_slices():
            pltpu.sync_copy(scbs[0].at[pl.ds(0, r)],
                            acc.at[pl.ds(base0 + off, r)])

        @pl.when(sid == _NT - 1)
        def _():
            pltpu.sync_copy(scbs[0].at[pl.ds(0, 16)],
                            acc.at[pl.ds(_NT * _RPT, 16)])
        plsc.subcore_barrier()

        def compute(par):
            xlb, xrb, scb = xlbs[par], xrbs[par], scbs[par]

            def unpack2(ref, r, b):
                w = ref[r, pl.ds(b * 16, 16)]
                return plsc.unpack(plsc.bitcast(w, jnp.bfloat16),
                                   format=plsc.PackFormat.INTERLEAVED,
                                   preferred_element_type=jnp.float32)

            def bshuf(v, s):
                return v + jnp.take_along_axis(v, lanes ^ s, axis=0)

            brev = (((lanes & 1) << 3) | ((lanes & 2) << 1)
                    | ((lanes & 4) >> 1) | ((lanes & 8) >> 3))
            m8 = lanes < 8
            m4 = (lanes & 4) == 0
            m2 = (lanes & 2) == 0
            m1 = (lanes & 1) == 0

            def group_body(g, _):
                base = g * 16
                accs = []
                for e in range(16):
                    r = base + e
                    accv = None
                    for b in range(2):
                        xla, xlb2 = unpack2(xlb, r, b)
                        xra, xrb2 = unpack2(xrb, r, b)
                        for k, (xv, rv) in enumerate(((xla, xra),
                                                      (xlb2, xrb2))):
                            s = xv + rv
                            ev = jnp.maximum(s, 0.2 * s)
                            t = ev * att_vecs[2 * b + k]
                            accv = t if accv is None else accv + t
                    accs.append(accv)
                lvl = [bshuf(v, 8) for v in accs]
                lvl = [jnp.where(m8, lvl[2 * j], lvl[2 * j + 1])
                       for j in range(8)]
                lvl = [bshuf(v, 4) for v in lvl]
                lvl = [jnp.where(m4, lvl[2 * j], lvl[2 * j + 1])
                       for j in range(4)]
                lvl = [bshuf(v, 2) for v in lvl]
                lvl = [jnp.where(m2, lvl[2 * j], lvl[2 * j + 1])
                       for j in range(2)]
                lvl = [bshuf(v, 1) for v in lvl]
                p = jnp.exp(jnp.where(m1, lvl[0], lvl[1]))
                for e in range(16):
                    r = base + e
                    pev = jnp.take_along_axis(
                        p, jnp.full((16,), _BREV[e], jnp.int32), axis=0)
                    for b in range(2):
                        xla, xlb2 = unpack2(xlb, r, b)
                        scb[r, pl.ds(b * 32, 16)] = xla * pev
                        scb[r, pl.ds(b * 32 + 16, 16)] = xlb2 * pev
                plsc.store_scatter(
                    scb, [base + brev, jnp.full((16,), _C, jnp.int32)], p)
                return 0

            lax.fori_loop(0, _K // 16, group_body, 0)

        issue(0, h, 0)

        def dbl_body(i, _):
            c0 = 2 * i
            issue(c0 + 1, h, 1)
            wait_gather(c0, h, 0)

            @pl.when(i > 0)
            def _():
                wait_scatter(c0 - 2, 0)
            compute(0)
            scatter(c0, 0)
            issue(c0 + 2, h, 0)
            wait_gather(c0 + 1, h, 1)

            @pl.when(i > 0)
            def _():
                wait_scatter(c0 - 1, 1)
            compute(1)
            scatter(c0 + 1, 1)
            return 0

        lax.fori_loop(0, (_NCH - 1) // 2, dbl_body, 0)
        wait_gather(_NCH - 1, h, 0)
        wait_scatter(_NCH - 3, 0)
        compute(0)
        scatter(_NCH - 1, 0)
        wait_scatter(_NCH - 2, 1)
        wait_scatter(_NCH - 1, 0)
        plsc.subcore_barrier()

        for off, r in _acc_slices():
            pltpu.sync_copy(acc.at[pl.ds(base0 + off, r)],
                            scbs[0].at[pl.ds(0, r)])
            pltpu.sync_copy(scbs[0].at[pl.ds(0, r)],
                            out_hbm.at[cid, h, pl.ds(base0 + off, r)])

        @pl.when(sid == _NT - 1)
        def _():
            pltpu.sync_copy(acc.at[pl.ds(_NT * _RPT, 16)],
                            scbs[0].at[pl.ds(0, 16)])
            pltpu.sync_copy(scbs[0].at[pl.ds(0, 16)],
                            out_hbm.at[cid, h, pl.ds(_NT * _RPT, 16)])


_edge_kernel = functools.partial(
    pl.kernel,
    out_type=jax.ShapeDtypeStruct((2, _H, _N, _ROW), jnp.float32),
    mesh=plsc.VectorSubcoreMesh(core_axis_name="c", subcore_axis_name="s"),
    compiler_params=pltpu.CompilerParams(use_tc_tiling_on_sc=False,
                                         needs_layout_passes=False),
    scratch_types=[
        pltpu.VMEM((_NCH, _K), jnp.int32),
        pltpu.VMEM((_NCH, _K), jnp.int32),
        pltpu.VMEM((_K, _C // 2), jnp.int32),
        pltpu.VMEM((_K, _C // 2), jnp.int32),
        pltpu.VMEM((_K, _C // 2), jnp.int32),
        pltpu.VMEM((_K, _C // 2), jnp.int32),
        pltpu.VMEM((_K, _ROW), jnp.float32),
        pltpu.VMEM((_K, _ROW), jnp.float32),
        pltpu.VMEM((_HC,), jnp.float32),
        pltpu.VMEM_SHARED((_N, _ROW), jnp.float32),
        pltpu.SemaphoreType.DMA,
        pltpu.SemaphoreType.DMA,
        pltpu.SemaphoreType.DMA,
        pltpu.SemaphoreType.DMA,
        pltpu.SemaphoreType.DMA,
        pltpu.SemaphoreType.DMA,
    ],
)(_edge_body)


def _post_body(xl0_ref, xl1_ref, xl2_ref, xr0_ref, xr1_ref, xr2_ref, acc_ref,
               gamma_ref, att_ref, bias_ref, wsem_ref, bsem_ref, qsem_ref,
               mg_ref, z_ref):
    xls = [xl0_ref[...], xl1_ref[...], xl2_ref[...]]
    xrs = [xr0_ref[...], xr1_ref[...], xr2_ref[...]]
    sps = []
    for h in range(_H):
        s = xls[h] + xrs[h]
        e = jnp.where(s >= 0.0, s, 0.2 * s)
        ea = e * att_ref[:, h * _C:(h + 1) * _C]
        sps.append(jnp.exp(jnp.sum(ea, axis=1, keepdims=True)))

    outs = []
    betas = []
    for m in range(2):
        parts = []
        for h in range(_H):
            accmh = acc_ref[m, h]
            sp = sps[h]
            den = accmh[:, _C][:, None] + sp
            numh = accmh[:, :_C] + sp * xls[h]
            parts.append(numh / den)
        out = jnp.concatenate(parts, axis=1) + bias_ref[...]
        out = jnp.where(out >= 0.0, out, 0.01 * out)
        w = jnp.tanh(jnp.dot(out, wsem_ref[...],
                             preferred_element_type=jnp.float32)
                     + bsem_ref[...])
        beta = jnp.sum(w * qsem_ref[...], axis=1, keepdims=True)
        outs.append(out)
        betas.append(beta)

    bmax = jnp.maximum(betas[0], betas[1])
    e0 = jnp.exp(betas[0] - bmax)
    e1 = jnp.exp(betas[1] - bmax)
    z = (e0 * outs[0] + e1 * outs[1]) / (e0 + e1)
    z_ref[...] = z + mg_ref[...] * gamma_ref[...]


def _post(xlh, xrh, acc, gamma, att_flat, bias_g, wsem, bsem, qsem, mg):
    return pl.pallas_call(
        _post_body,
        grid=(_N // _PBB,),
        in_specs=(
            [pl.BlockSpec((_PBB, _C), lambda i: (i, 0))] * (2 * _H)
            + [
                pl.BlockSpec((2, _H, _PBB, _ROW), lambda i: (0, 0, i, 0)),
                pl.BlockSpec((_PBB, 1), lambda i: (i, 0)),
                pl.BlockSpec((1, _HC), lambda i: (0, 0)),
                pl.BlockSpec((1, _HC), lambda i: (0, 0)),
                pl.BlockSpec((_HC, _SEM), lambda i: (0, 0)),
                pl.BlockSpec((1, _SEM), lambda i: (0, 0)),
                pl.BlockSpec((1, _SEM), lambda i: (0, 0)),
                pl.BlockSpec((1, _HC), lambda i: (0, 0)),
            ]
        ),
        out_specs=pl.BlockSpec((_PBB, _HC), lambda i: (i, 0)),
        out_shape=jax.ShapeDtypeStruct((_N, _HC), jnp.float32),
    )(*xlh, *xrh, acc, gamma, att_flat, bias_g, wsem, bsem, qsem, mg)


def _pack_rows(a):
    blocks = []
    for b in range(2):
        pair = jnp.stack([a[:, b * 32:b * 32 + 16],
                          a[:, b * 32 + 16:b * 32 + 32]], axis=-1)
        blocks.append(jax.lax.bitcast_convert_type(
            pair.astype(jnp.bfloat16), jnp.int32))
    return jnp.concatenate(blocks, axis=1)


@jax.jit
def kernel(x, edge_index0, edge_index1, mg_x, gamma, Wl, Wr, att, bias_g,
           W_sem, b_sem, q_sem):
    proj = _project(x, Wl, Wr)
    xlh, xrh = proj[:_H], proj[_H:]
    xlp = [_pack_rows(a) for a in xlh]
    xrp = [_pack_rows(a) for a in xrh]
    att_flat = att.reshape(_HC)
    ei0r = edge_index0.reshape(2, _NT, _NCH, _K)
    ei1r = edge_index1.reshape(2, _NT, _NCH, _K)
    acc = _edge_kernel(ei0r, ei1r, *xlp, *xrp, att_flat)
    return _post(xlh, xrh, acc, gamma.reshape(_N, 1), att_flat.reshape(1, _HC),
                 bias_g.reshape(1, _HC), W_sem.reshape(_HC, _SEM),
                 b_sem.reshape(1, _SEM), q_sem.reshape(1, _SEM), mg_x)


# speedup vs baseline: 1.0173x; 1.0173x over previous
import functools

import jax
import jax.numpy as jnp
from jax import lax
from jax.experimental import pallas as pl
from jax.experimental.pallas import tpu as pltpu
from jax.experimental.pallas import tpu_sc as plsc

_N = 10000
_F_IN = 128
_H = 3
_C = 64
_HC = _H * _C
_SEM = 128
_E = 160000
_ROW = 72
_K = 80
_NT = 16
_EPT = _E // _NT
_NCH = _EPT // _K
_RPT = 624
_MMB = 1000
_PBB = 1000


def _mm_body(x_ref, wl_ref, wr_ref, *out_refs):
    x = x_ref[...]
    xl = jnp.dot(x, wl_ref[...], preferred_element_type=jnp.float32)
    xr = jnp.dot(x, wr_ref[...], preferred_element_type=jnp.float32)
    for h in range(_H):
        out_refs[h][...] = xl[:, h * _C:(h + 1) * _C]
        out_refs[_H + h][...] = xr[:, h * _C:(h + 1) * _C]


def _project(x, Wl, Wr):
    return pl.pallas_call(
        _mm_body,
        grid=(_N // _MMB,),
        in_specs=[
            pl.BlockSpec((_MMB, _F_IN), lambda i: (i, 0)),
            pl.BlockSpec((_F_IN, _HC), lambda i: (0, 0)),
            pl.BlockSpec((_F_IN, _HC), lambda i: (0, 0)),
        ],
        out_specs=[pl.BlockSpec((_MMB, _C), lambda i: (i, 0))] * (2 * _H),
        out_shape=[jax.ShapeDtypeStruct((_N, _C), jnp.float32)] * (2 * _H),
    )(x, Wl, Wr)


def _acc_slices():
    out = []
    off = 0
    while off < _RPT:
        r = min(_K, _RPT - off)
        out.append((off, r))
        off += r
    return out


def _edge_body(ei0, ei1, xl0, xl1, xl2, xr0, xr1, xr2, att_hbm, out_hbm,
               srcb, dstb, xlb0, xlb1, xrb0, xrb1, scb0, scb1,
               att_v, acc, sl0, sl1, sr0, sr1, ss0, ss1):
    cid = lax.axis_index("c")
    sid = lax.axis_index("s")
    xls = (xl0, xl1, xl2)
    xrs = (xr0, xr1, xr2)
    xlbs = (xlb0, xlb1)
    xrbs = (xrb0, xrb1)
    scbs = (scb0, scb1)
    sls = (sl0, sl1)
    srs = (sr0, sr1)
    sss = (ss0, ss1)

    pltpu.sync_copy(att_hbm, att_v)

    @pl.when(cid == 0)
    def _():
        pltpu.sync_copy(ei0.at[0, sid], srcb)
        pltpu.sync_copy(ei0.at[1, sid], dstb)

    @pl.when(cid != 0)
    def _():
        pltpu.sync_copy(ei1.at[0, sid], srcb)
        pltpu.sync_copy(ei1.at[1, sid], dstb)

    base0 = sid * _RPT
    lanes = lax.iota(jnp.int32, 16)

    def issue(i, h, par):
        pltpu.async_copy(xls[h].at[srcb.at[i]], xlbs[par], sls[par])
        pltpu.async_copy(xrs[h].at[dstb.at[i]], xrbs[par], srs[par])

    def wait_gather(i, h, par):
        pltpu.make_async_copy(xls[h].at[srcb.at[i]], xlbs[par],
                              sls[par]).wait()
        pltpu.make_async_copy(xrs[h].at[dstb.at[i]], xrbs[par],
                              srs[par]).wait()

    def scatter(i, par):
        pltpu.async_copy(scbs[par], acc.at[dstb.at[i]], sss[par], add=True)

    def wait_scatter(i, par):
        pltpu.make_async_copy(scbs[par], acc.at[dstb.at[i]],
                              sss[par]).wait()

    for h in range(_H):
        att_vecs = [att_v[pl.ds(h * _C + k * 16, 16)] for k in range(_C // 16)]

        def _zero_row16(r, _):
            for scb in scbs:
                for c in range(4):
                    scb[r, pl.ds(c * 16, 16)] = jnp.zeros((16,), jnp.float32)
                scb[r, pl.ds(56, 16)] = jnp.zeros((16,), jnp.float32)
            return 0
        lax.fori_loop(0, _K, _zero_row16, 0)

        for off, r in _acc_slices():
            pltpu.sync_copy(scbs[0].at[pl.ds(0, r)],
                            acc.at[pl.ds(base0 + off, r)])

        @pl.when(sid == _NT - 1)
        def _():
            pltpu.sync_copy(scbs[0].at[pl.ds(0, 16)],
                            acc.at[pl.ds(_NT * _RPT, 16)])
        plsc.subcore_barrier()

        def compute(par):
            xlb, xrb, scb = xlbs[par], xrbs[par], scbs[par]

            def unpack2(ref, r, b):
                w = ref[r, pl.ds(b * 16, 16)]
                return plsc.unpack(plsc.bitcast(w, jnp.bfloat16),
                                   format=plsc.PackFormat.INTERLEAVED,
                                   preferred_element_type=jnp.float32)

            def group_body(g, _):
                base = g * 16
                pvec = jnp.zeros((16,), jnp.float32)
                for e in range(16):
                    r = base + e
                    accv = None
                    for b in range(2):
                        xla, xlb2 = unpack2(xlb, r, b)
                        xra, xrb2 = unpack2(xrb, r, b)
                        for k, (xv, rv) in enumerate(((xla, xra),
                                                      (xlb2, xrb2))):
                            s = xv + rv
                            ev = jnp.maximum(s, 0.2 * s)
                            t = ev * att_vecs[2 * b + k]
                            accv = t if accv is None else accv + t
                    pvec = jnp.where(lanes == e, jnp.sum(accv), pvec)
                p = jnp.exp(pvec)
                for e in range(16):
                    r = base + e
                    pe = p[e]
                    for b in range(2):
                        xla, xlb2 = unpack2(xlb, r, b)
                        scb[r, pl.ds(b * 32, 16)] = xla * pe
                        scb[r, pl.ds(b * 32 + 16, 16)] = xlb2 * pe
                plsc.store_scatter(
                    scb, [base + lanes, jnp.full((16,), _C, jnp.int32)], p)
                return 0

            lax.fori_loop(0, _K // 16, group_body, 0)

        issue(0, h, 0)

        def dbl_body(i, _):
            c0 = 2 * i
            issue(c0 + 1, h, 1)
            wait_gather(c0, h, 0)

            @pl.when(i > 0)
            def _():
                wait_scatter(c0 - 2, 0)
            compute(0)
            scatter(c0, 0)
            issue(c0 + 2, h, 0)
            wait_gather(c0 + 1, h, 1)

            @pl.when(i > 0)
            def _():
                wait_scatter(c0 - 1, 1)
            compute(1)
            scatter(c0 + 1, 1)
            return 0

        lax.fori_loop(0, (_NCH - 1) // 2, dbl_body, 0)
        wait_gather(_NCH - 1, h, 0)
        wait_scatter(_NCH - 3, 0)
        compute(0)
        scatter(_NCH - 1, 0)
        wait_scatter(_NCH - 2, 1)
        wait_scatter(_NCH - 1, 0)
        plsc.subcore_barrier()

        for off, r in _acc_slices():
            pltpu.sync_copy(acc.at[pl.ds(base0 + off, r)],
                            scbs[0].at[pl.ds(0, r)])
            pltpu.sync_copy(scbs[0].at[pl.ds(0, r)],
                            out_hbm.at[cid, h, pl.ds(base0 + off, r)])

        @pl.when(sid == _NT - 1)
        def _():
            pltpu.sync_copy(acc.at[pl.ds(_NT * _RPT, 16)],
                            scbs[0].at[pl.ds(0, 16)])
            pltpu.sync_copy(scbs[0].at[pl.ds(0, 16)],
                            out_hbm.at[cid, h, pl.ds(_NT * _RPT, 16)])


_edge_kernel = functools.partial(
    pl.kernel,
    out_type=jax.ShapeDtypeStruct((2, _H, _N, _ROW), jnp.float32),
    mesh=plsc.VectorSubcoreMesh(core_axis_name="c", subcore_axis_name="s"),
    compiler_params=pltpu.CompilerParams(use_tc_tiling_on_sc=False,
                                         needs_layout_passes=False),
    scratch_types=[
        pltpu.VMEM((_NCH, _K), jnp.int32),
        pltpu.VMEM((_NCH, _K), jnp.int32),
        pltpu.VMEM((_K, _C // 2), jnp.int32),
        pltpu.VMEM((_K, _C // 2), jnp.int32),
        pltpu.VMEM((_K, _C // 2), jnp.int32),
        pltpu.VMEM((_K, _C // 2), jnp.int32),
        pltpu.VMEM((_K, _ROW), jnp.float32),
        pltpu.VMEM((_K, _ROW), jnp.float32),
        pltpu.VMEM((_HC,), jnp.float32),
        pltpu.VMEM_SHARED((_N, _ROW), jnp.float32),
        pltpu.SemaphoreType.DMA,
        pltpu.SemaphoreType.DMA,
        pltpu.SemaphoreType.DMA,
        pltpu.SemaphoreType.DMA,
        pltpu.SemaphoreType.DMA,
        pltpu.SemaphoreType.DMA,
    ],
)(_edge_body)


def _post_body(xl0_ref, xl1_ref, xl2_ref, xr0_ref, xr1_ref, xr2_ref, acc_ref,
               gamma_ref, att_ref, bias_ref, wsem_ref, bsem_ref, qsem_ref,
               mg_ref, z_ref):
    xls = [xl0_ref[...], xl1_ref[...], xl2_ref[...]]
    xrs = [xr0_ref[...], xr1_ref[...], xr2_ref[...]]
    sps = []
    for h in range(_H):
        s = xls[h] + xrs[h]
        e = jnp.where(s >= 0.0, s, 0.2 * s)
        ea = e * att_ref[:, h * _C:(h + 1) * _C]
        sps.append(jnp.exp(jnp.sum(ea, axis=1, keepdims=True)))

    outs = []
    betas = []
    for m in range(2):
        parts = []
        for h in range(_H):
            accmh = acc_ref[m, h]
            sp = sps[h]
            den = accmh[:, _C][:, None] + sp
            numh = accmh[:, :_C] + sp * xls[h]
            parts.append(numh / den)
        out = jnp.concatenate(parts, axis=1) + bias_ref[...]
        out = jnp.where(out >= 0.0, out, 0.01 * out)
        w = jnp.tanh(jnp.dot(out, wsem_ref[...],
                             preferred_element_type=jnp.float32)
                     + bsem_ref[...])
        beta = jnp.sum(w * qsem_ref[...], axis=1, keepdims=True)
        outs.append(out)
        betas.append(beta)

    bmax = jnp.maximum(betas[0], betas[1])
    e0 = jnp.exp(betas[0] - bmax)
    e1 = jnp.exp(betas[1] - bmax)
    z = (e0 * outs[0] + e1 * outs[1]) / (e0 + e1)
    z_ref[...] = z + mg_ref[...] * gamma_ref[...]


def _post(xlh, xrh, acc, gamma, att_flat, bias_g, wsem, bsem, qsem, mg):
    return pl.pallas_call(
        _post_body,
        grid=(_N // _PBB,),
        in_specs=(
            [pl.BlockSpec((_PBB, _C), lambda i: (i, 0))] * (2 * _H)
            + [
                pl.BlockSpec((2, _H, _PBB, _ROW), lambda i: (0, 0, i, 0)),
                pl.BlockSpec((_PBB, 1), lambda i: (i, 0)),
                pl.BlockSpec((1, _HC), lambda i: (0, 0)),
                pl.BlockSpec((1, _HC), lambda i: (0, 0)),
                pl.BlockSpec((_HC, _SEM), lambda i: (0, 0)),
                pl.BlockSpec((1, _SEM), lambda i: (0, 0)),
                pl.BlockSpec((1, _SEM), lambda i: (0, 0)),
                pl.BlockSpec((1, _HC), lambda i: (0, 0)),
            ]
        ),
        out_specs=pl.BlockSpec((_PBB, _HC), lambda i: (i, 0)),
        out_shape=jax.ShapeDtypeStruct((_N, _HC), jnp.float32),
    )(*xlh, *xrh, acc, gamma, att_flat, bias_g, wsem, bsem, qsem, mg)


def _pack_rows(a):
    blocks = []
    for b in range(2):
        pair = jnp.stack([a[:, b * 32:b * 32 + 16],
                          a[:, b * 32 + 16:b * 32 + 32]], axis=-1)
        blocks.append(jax.lax.bitcast_convert_type(
            pair.astype(jnp.bfloat16), jnp.int32))
    return jnp.concatenate(blocks, axis=1)


@jax.jit
def kernel(x, edge_index0, edge_index1, mg_x, gamma, Wl, Wr, att, bias_g,
           W_sem, b_sem, q_sem):
    proj = _project(x, Wl, Wr)
    xlh, xrh = proj[:_H], proj[_H:]
    xlp = [_pack_rows(a) for a in xlh]
    xrp = [_pack_rows(a) for a in xrh]
    att_flat = att.reshape(_HC)
    ei0r = edge_index0.reshape(2, _NT, _NCH, _K)
    ei1r = edge_index1.reshape(2, _NT, _NCH, _K)
    acc = _edge_kernel(ei0r, ei1r, *xlp, *xrp, att_flat)
    return _post(xlh, xrh, acc, gamma.reshape(_N, 1), att_flat.reshape(1, _HC),
                 bias_g.reshape(1, _HC), W_sem.reshape(_HC, _SEM),
                 b_sem.reshape(1, _SEM), q_sem.reshape(1, _SEM), mg_x)
